# Initial kernel scaffold; baseline (speedup 1.0000x reference)
import jax
import jax.numpy as jnp
from jax.experimental import pallas as pl


def kernel(node_attr, edge_input, edge_index, conv_w, conv_b, W1, b1, W2, b2, W3, b3):
    raise NotImplementedError("write your pallas kernel here")


# speedup vs baseline: 1.6067x; 1.6067x over previous
import functools

import jax
import jax.numpy as jnp
from jax import lax
from jax.experimental import pallas as pl
from jax.experimental.pallas import tpu as pltpu
from jax.experimental.pallas import tpu_sc as plsc

N = 10000
E = 160000
H = 256


_BN = 1000


def _node_emb_body(x_ref, w_ref, b_ref, o_ref):
    x = x_ref[...]
    acc = x[:, 0:H] * w_ref[0]
    acc += x[:, H:2 * H] * w_ref[1]
    acc += x[:, 2 * H:3 * H] * w_ref[2]
    acc += x[:, 3 * H:4 * H] * w_ref[3]
    o_ref[...] = acc + b_ref[0]


def _node_emb_tc(x, conv_w, conv_b):
    return pl.pallas_call(
        _node_emb_body,
        grid=(N // _BN,),
        in_specs=[
            pl.BlockSpec((_BN, 4 * H), lambda i: (i, 0)),
            pl.BlockSpec(memory_space=pltpu.SMEM),
            pl.BlockSpec(memory_space=pltpu.SMEM),
        ],
        out_specs=pl.BlockSpec((_BN, H), lambda i: (i, 0)),
        out_shape=jax.ShapeDtypeStruct((N, H), jnp.float32),
    )(x, conv_w, conv_b)


_NW = 32
_C = 40
_CHUNKS = E // _C
_PER_W = _CHUNKS // _NW


def _sc_gather(tbl, row, col):
    mesh = plsc.VectorSubcoreMesh(core_axis_name="c", subcore_axis_name="s")
    dt = tbl.dtype

    @functools.partial(
        pl.kernel,
        out_type=(
            jax.ShapeDtypeStruct((E, H), dt),
            jax.ShapeDtypeStruct((E, H), dt),
        ),
        mesh=mesh,
        scratch_types=[
            pltpu.VMEM((_C,), jnp.int32),
            pltpu.VMEM((_C,), jnp.int32),
            pltpu.VMEM((_C, H), dt),
            pltpu.VMEM((_C, H), dt),
            pltpu.SemaphoreType.DMA,
            pltpu.SemaphoreType.DMA,
        ],
    )
    def k(tbl_hbm, row_hbm, col_hbm, osrc_hbm, odst_hbm,
          idx_r, idx_c, rows_r, rows_c, sem_r, sem_c):
        wid = lax.axis_index("c") * 16 + lax.axis_index("s")

        @pl.loop(0, _PER_W)
        def _(i):
            base = (wid * _PER_W + i) * _C
            pltpu.sync_copy(row_hbm.at[pl.ds(base, _C)], idx_r)
            pltpu.sync_copy(col_hbm.at[pl.ds(base, _C)], idx_c)
            cp_r = pltpu.async_copy(tbl_hbm.at[idx_r], rows_r, sem_r)
            cp_c = pltpu.async_copy(tbl_hbm.at[idx_c], rows_c, sem_c)
            cp_r.wait()
            pltpu.sync_copy(rows_r, osrc_hbm.at[pl.ds(base, _C)])
            cp_c.wait()
            pltpu.sync_copy(rows_c, odst_hbm.at[pl.ds(base, _C)])

    return k(tbl, row, col)


_BE = 2000


def _mlp_body(src_ref, dst_ref, edge_ref, w1a_ref, w1b_ref, w1c_ref, b1_ref,
              w2_ref, b2_ref, w3_ref, b3_ref, o_ref):
    f32 = jnp.float32
    h = jnp.dot(src_ref[...], w1a_ref[...], preferred_element_type=f32)
    h += jnp.dot(dst_ref[...], w1b_ref[...], preferred_element_type=f32)
    h += jnp.dot(edge_ref[...], w1c_ref[...], preferred_element_type=f32)
    h = jnp.maximum(h + b1_ref[...], 0.0)
    h = jnp.dot(h, w2_ref[...], preferred_element_type=f32) + b2_ref[...]
    h = jnp.maximum(h, 0.0)
    o_ref[...] = jnp.dot(h, w3_ref[...], preferred_element_type=f32) + b3_ref[...]


def _mlp_tc(gsrc, gdst, edge_input, w1a, w1b, w1c, b1, w2, b2, w3, b3):
    d1, d2, d3 = w1a.shape[1], w2.shape[1], w3.shape[1]
    full = lambda shape: pl.BlockSpec(shape, lambda i: tuple(0 for _ in shape))
    return pl.pallas_call(
        _mlp_body,
        grid=(E // _BE,),
        in_specs=[
            pl.BlockSpec((_BE, H), lambda i: (i, 0)),
            pl.BlockSpec((_BE, H), lambda i: (i, 0)),
            pl.BlockSpec((_BE, H), lambda i: (i, 0)),
            full(w1a.shape), full(w1b.shape), full(w1c.shape), full((1, d1)),
            full(w2.shape), full((1, d2)),
            full(w3.shape), full((1, d3)),
        ],
        out_specs=pl.BlockSpec((_BE, d3), lambda i: (i, 0)),
        out_shape=jax.ShapeDtypeStruct((E, d3), jnp.float32),
    )(gsrc, gdst, edge_input, w1a, w1b, w1c, b1.reshape(1, d1),
      w2, b2.reshape(1, d2), w3, b3.reshape(1, d3))


def kernel(node_attr, edge_input, edge_index, conv_w, conv_b,
           W1, b1, W2, b2, W3, b3):
    x = node_attr.reshape(N, 4 * H)
    tbl = _node_emb_tc(x, conv_w, conv_b)
    row = edge_index[0]
    col = edge_index[1]
    gsrc, gdst = _sc_gather(tbl, row, col)
    w1a, w1b, w1c = W1[:H], W1[H:2 * H], W1[2 * H:]
    return _mlp_tc(gsrc, gdst, edge_input, w1a, w1b, w1c, b1, W2, b2, W3, b3)


# speedup vs baseline: 1.8178x; 1.1314x over previous
import functools

import jax
import jax.numpy as jnp
from jax import lax
from jax.experimental import pallas as pl
from jax.experimental.pallas import tpu as pltpu
from jax.experimental.pallas import tpu_sc as plsc

N = 10000
E = 160000
H = 256
HP = H // 2


_BN = 1000


def _node_emb_body(x_ref, w_ref, b_ref, o_ref):
    x = x_ref[...]
    acc = x[:, 0:H] * w_ref[0]
    acc += x[:, H:2 * H] * w_ref[1]
    acc += x[:, 2 * H:3 * H] * w_ref[2]
    acc += x[:, 3 * H:4 * H] * w_ref[3]
    acc = acc + b_ref[0]
    rnd = lambda v: lax.bitcast_convert_type(
        v.astype(jnp.bfloat16).astype(jnp.float32), jnp.int32)
    lo = lax.shift_right_logical(rnd(acc[:, :HP]), 16)
    hi = jnp.bitwise_and(rnd(acc[:, HP:]), jnp.int32(-65536))
    o_ref[...] = jnp.bitwise_or(lo, hi)


def _node_emb_tc(x, conv_w, conv_b):
    return pl.pallas_call(
        _node_emb_body,
        grid=(N // _BN,),
        in_specs=[
            pl.BlockSpec((_BN, 4 * H), lambda i: (i, 0)),
            pl.BlockSpec(memory_space=pltpu.SMEM),
            pl.BlockSpec(memory_space=pltpu.SMEM),
        ],
        out_specs=pl.BlockSpec((_BN, HP), lambda i: (i, 0)),
        out_shape=jax.ShapeDtypeStruct((N, HP), jnp.int32),
    )(x, conv_w, conv_b)


_NW = 32
_C = 40
_CHUNKS = E // _C
_PER_W = _CHUNKS // _NW


def _sc_gather(tbl, edge_index_flat):
    mesh = plsc.VectorSubcoreMesh(core_axis_name="c", subcore_axis_name="s")
    dt = tbl.dtype

    @functools.partial(
        pl.kernel,
        out_type=(
            jax.ShapeDtypeStruct((E, HP), dt),
            jax.ShapeDtypeStruct((E, HP), dt),
        ),
        mesh=mesh,
        scratch_types=[
            pltpu.VMEM((_C,), jnp.int32),
            pltpu.VMEM((_C,), jnp.int32),
            pltpu.VMEM((_C, HP), dt),
            pltpu.VMEM((_C, HP), dt),
            pltpu.SemaphoreType.DMA,
            pltpu.SemaphoreType.DMA,
        ],
    )
    def k(tbl_hbm, ei_hbm, osrc_hbm, odst_hbm,
          idx_r, idx_c, rows_r, rows_c, sem_r, sem_c):
        wid = lax.axis_index("c") * 16 + lax.axis_index("s")

        @pl.loop(0, _PER_W)
        def _(i):
            base = (wid * _PER_W + i) * _C
            pltpu.sync_copy(ei_hbm.at[pl.ds(base, _C)], idx_r)
            pltpu.sync_copy(ei_hbm.at[pl.ds(E + base, _C)], idx_c)
            cp_r = pltpu.async_copy(tbl_hbm.at[idx_r], rows_r, sem_r)
            cp_c = pltpu.async_copy(tbl_hbm.at[idx_c], rows_c, sem_c)
            cp_r.wait()
            pltpu.sync_copy(rows_r, osrc_hbm.at[pl.ds(base, _C)])
            cp_c.wait()
            pltpu.sync_copy(rows_c, odst_hbm.at[pl.ds(base, _C)])

    return k(tbl, edge_index_flat)


_BE = 2000


def _unpack(x):
    f32 = jnp.float32
    lo = lax.bitcast_convert_type(lax.shift_left(x, 16), f32)
    hi = lax.bitcast_convert_type(jnp.bitwise_and(x, jnp.int32(-65536)), f32)
    return lo, hi


def _mlp_body(src_ref, dst_ref, edge_ref, w1_ref, b1_ref,
              w2_ref, b2_ref, w3_ref, b3_ref, o_ref):
    f32 = jnp.float32
    bf16 = jnp.bfloat16
    src_lo, src_hi = _unpack(src_ref[...])
    dst_lo, dst_hi = _unpack(dst_ref[...])
    x = jnp.concatenate(
        [src_lo, src_hi, dst_lo, dst_hi, edge_ref[...]], axis=1).astype(bf16)
    h = jnp.dot(x, w1_ref[...], preferred_element_type=f32)
    h = jnp.maximum(h + b1_ref[...], 0.0).astype(bf16)
    h = jnp.dot(h, w2_ref[...], preferred_element_type=f32) + b2_ref[...]
    h = jnp.maximum(h, 0.0).astype(bf16)
    o_ref[...] = jnp.dot(h, w3_ref[...], preferred_element_type=f32) + b3_ref[...]


def _mlp_tc(gsrc, gdst, edge_input, w1, b1, w2, b2, w3, b3):
    d1, d2, d3 = w1.shape[1], w2.shape[1], w3.shape[1]
    full = lambda shape: pl.BlockSpec(shape, lambda i: tuple(0 for _ in shape))
    return pl.pallas_call(
        _mlp_body,
        grid=(E // _BE,),
        in_specs=[
            pl.BlockSpec((_BE, HP), lambda i: (i, 0)),
            pl.BlockSpec((_BE, HP), lambda i: (i, 0)),
            pl.BlockSpec((_BE, H), lambda i: (i, 0)),
            full(w1.shape), full((1, d1)),
            full(w2.shape), full((1, d2)),
            full(w3.shape), full((1, d3)),
        ],
        out_specs=pl.BlockSpec((_BE, d3), lambda i: (i, 0)),
        out_shape=jax.ShapeDtypeStruct((E, d3), jnp.float32),
    )(gsrc, gdst, edge_input, w1, b1.reshape(1, d1),
      w2, b2.reshape(1, d2), w3, b3.reshape(1, d3))


def kernel(node_attr, edge_input, edge_index, conv_w, conv_b,
           W1, b1, W2, b2, W3, b3):
    x = node_attr.reshape(N, 4 * H)
    tbl = _node_emb_tc(x, conv_w, conv_b)
    gsrc, gdst = _sc_gather(tbl, edge_index.reshape(2 * E))
    bf16 = jnp.bfloat16
    return _mlp_tc(gsrc, gdst, edge_input, W1.astype(bf16), b1,
                   W2.astype(bf16), b2, W3.astype(bf16), b3)


# speedup vs baseline: 2.2337x; 1.2288x over previous
import functools

import jax
import jax.numpy as jnp
from jax import lax
from jax.experimental import pallas as pl
from jax.experimental.pallas import tpu as pltpu
from jax.experimental.pallas import tpu_sc as plsc

N = 10000
E = 160000
H = 256
HP = H // 2


_BN = 1000


def _node_emb_body(x_ref, w_ref, b_ref, o_ref):
    x = x_ref[...]
    acc = x[:, 0:H] * w_ref[0]
    acc += x[:, H:2 * H] * w_ref[1]
    acc += x[:, 2 * H:3 * H] * w_ref[2]
    acc += x[:, 3 * H:4 * H] * w_ref[3]
    acc = acc + b_ref[0]
    rnd = lambda v: lax.bitcast_convert_type(
        v.astype(jnp.bfloat16).astype(jnp.float32), jnp.int32)
    lo = lax.shift_right_logical(rnd(acc[:, :HP]), 16)
    hi = jnp.bitwise_and(rnd(acc[:, HP:]), jnp.int32(-65536))
    o_ref[...] = jnp.bitwise_or(lo, hi)


def _node_emb_tc(x, conv_w, conv_b):
    return pl.pallas_call(
        _node_emb_body,
        grid=(N // _BN,),
        in_specs=[
            pl.BlockSpec((_BN, 4 * H), lambda i: (i, 0)),
            pl.BlockSpec(memory_space=pltpu.SMEM),
            pl.BlockSpec(memory_space=pltpu.SMEM),
        ],
        out_specs=pl.BlockSpec((_BN, HP), lambda i: (i, 0)),
        out_shape=jax.ShapeDtypeStruct((N, HP), jnp.int32),
    )(x, conv_w, conv_b)


_NW = 32
_C = 128
_CHUNKS = E // _C
_REM = _CHUNKS % _NW


def _sc_gather(tbl, edge_index):
    mesh = plsc.VectorSubcoreMesh(core_axis_name="c", subcore_axis_name="s")
    dt = tbl.dtype

    @functools.partial(
        pl.kernel,
        out_type=(
            jax.ShapeDtypeStruct((E, HP), dt),
            jax.ShapeDtypeStruct((E, HP), dt),
        ),
        mesh=mesh,
        scratch_types=[
            pltpu.VMEM((2, _C), jnp.int32),
            pltpu.VMEM((_C, HP), dt),
            pltpu.VMEM((_C, HP), dt),
            pltpu.SemaphoreType.DMA,
            pltpu.SemaphoreType.DMA,
        ],
    )
    def k(tbl_hbm, ei_hbm, osrc_hbm, odst_hbm,
          idx_v, rows_r, rows_c, sem_r, sem_c):
        wid = lax.axis_index("c") * 16 + lax.axis_index("s")
        start = wid * (_CHUNKS // _NW) + jnp.minimum(wid, _REM)
        my_n = jnp.where(wid < _REM, _CHUNKS // _NW + 1, _CHUNKS // _NW)

        def body(i, carry):
            base = pl.multiple_of((start + i) * _C, _C)
            pltpu.sync_copy(ei_hbm.at[:, pl.ds(base, _C)], idx_v)
            cp_r = pltpu.async_copy(tbl_hbm.at[idx_v.at[0]], rows_r, sem_r)
            cp_c = pltpu.async_copy(tbl_hbm.at[idx_v.at[1]], rows_c, sem_c)
            cp_r.wait()
            pltpu.sync_copy(rows_r, osrc_hbm.at[pl.ds(base, _C)])
            cp_c.wait()
            pltpu.sync_copy(rows_c, odst_hbm.at[pl.ds(base, _C)])
            return carry

        lax.fori_loop(0, my_n, body, 0)

    return k(tbl, edge_index)


_BE = 2000


def _unpack(x):
    f32 = jnp.float32
    lo = lax.bitcast_convert_type(lax.shift_left(x, 16), f32)
    hi = lax.bitcast_convert_type(jnp.bitwise_and(x, jnp.int32(-65536)), f32)
    return lo, hi


def _mlp_body(src_ref, dst_ref, edge_ref, w1_ref, b1_ref,
              w2_ref, b2_ref, w3_ref, b3_ref, o_ref):
    f32 = jnp.float32
    bf16 = jnp.bfloat16
    src_lo, src_hi = _unpack(src_ref[...])
    dst_lo, dst_hi = _unpack(dst_ref[...])
    x = jnp.concatenate(
        [src_lo, src_hi, dst_lo, dst_hi, edge_ref[...]], axis=1).astype(bf16)
    h = jnp.dot(x, w1_ref[...], preferred_element_type=f32)
    h = jnp.maximum(h + b1_ref[...], 0.0).astype(bf16)
    h = jnp.dot(h, w2_ref[...], preferred_element_type=f32) + b2_ref[...]
    h = jnp.maximum(h, 0.0).astype(bf16)
    o_ref[...] = jnp.dot(h, w3_ref[...], preferred_element_type=f32) + b3_ref[...]


def _mlp_tc(gsrc, gdst, edge_input, w1, b1, w2, b2, w3, b3):
    d1, d2, d3 = w1.shape[1], w2.shape[1], w3.shape[1]
    full = lambda shape: pl.BlockSpec(shape, lambda i: tuple(0 for _ in shape))
    return pl.pallas_call(
        _mlp_body,
        grid=(E // _BE,),
        in_specs=[
            pl.BlockSpec((_BE, HP), lambda i: (i, 0)),
            pl.BlockSpec((_BE, HP), lambda i: (i, 0)),
            pl.BlockSpec((_BE, H), lambda i: (i, 0)),
            full(w1.shape), full((1, d1)),
            full(w2.shape), full((1, d2)),
            full(w3.shape), full((1, d3)),
        ],
        out_specs=pl.BlockSpec((_BE, d3), lambda i: (i, 0)),
        out_shape=jax.ShapeDtypeStruct((E, d3), jnp.float32),
    )(gsrc, gdst, edge_input, w1, b1.reshape(1, d1),
      w2, b2.reshape(1, d2), w3, b3.reshape(1, d3))


def kernel(node_attr, edge_input, edge_index, conv_w, conv_b,
           W1, b1, W2, b2, W3, b3):
    x = node_attr.reshape(N, 4 * H)
    tbl = _node_emb_tc(x, conv_w, conv_b)
    gsrc, gdst = _sc_gather(tbl, edge_index)
    bf16 = jnp.bfloat16
    return _mlp_tc(gsrc, gdst, edge_input, W1.astype(bf16), b1,
                   W2.astype(bf16), b2, W3.astype(bf16), b3)
